# Initial kernel scaffold; baseline (speedup 1.0000x reference)
import functools

import jax
import jax.numpy as jnp
from jax import lax
from jax.experimental import pallas as pl
from jax.experimental.pallas import tpu as pltpu
from jax.experimental.pallas import tpu_sc as plsc

N = 10000
E = 320000
D = 128
N_LAYERS = 3
ALPHA = sum(1.0 / (1 + k) for k in range(N_LAYERS))

NC, NS, L = 2, 16, 16
NW = NC * NS
EPW = E // NW
C = 80
NCHUNK = EPW // C
RPT = N // NS
ZR = 125
NZ = RPT // ZR

_mesh = plsc.VectorSubcoreMesh(
    core_axis_name="c", subcore_axis_name="s", num_cores=NC, num_subcores=NS
)


@functools.partial(
    pl.kernel,
    out_type=jax.ShapeDtypeStruct((NW, 2, N), jnp.float32),
    mesh=_mesh,
    scratch_types=[
        pltpu.VMEM((EPW,), jnp.int32),
        pltpu.VMEM((2, N), jnp.float32),
    ],
)
def _deg_kernel(edge_hbm, deg_hbm, ev, cnt):
    c = lax.axis_index("c")
    s = lax.axis_index("s")
    wid = c * NS + s
    base = wid * EPW
    zeros = jnp.zeros((L,), jnp.float32)
    ones = jnp.ones((L,), jnp.float32)

    def zero_body(g, _):
        cnt[0, pl.ds(g * L, L)] = zeros
        cnt[1, pl.ds(g * L, L)] = zeros
        return 0

    lax.fori_loop(0, N // L, zero_body, 0)

    for kind in (0, 1):
        pltpu.sync_copy(edge_hbm.at[kind, pl.ds(base, EPW)], ev)

        def acc_body(g, _, kind=kind):
            idx = ev[pl.ds(g * L, L)]
            plsc.addupdate_scatter(cnt.at[kind], [idx], ones)
            return 0

        lax.fori_loop(0, EPW // L, acc_body, 0)

    pltpu.sync_copy(cnt, deg_hbm.at[wid])


def _norm_body(emb_ref, deg_ref, m_ref, innorm_ref):
    out_deg = jnp.sum(deg_ref[:, 0, :], axis=0)
    in_deg = jnp.sum(deg_ref[:, 1, :], axis=0)
    out_norm = lax.rsqrt(jnp.maximum(out_deg, 1.0))
    innorm_ref[...] = lax.rsqrt(jnp.maximum(in_deg, 1.0))
    m_ref[...] = emb_ref[...] * out_norm[:, None]


_RB = 2000


def _norm_call(emb, degs):
    return pl.pallas_call(
        _norm_body,
        grid=(N // _RB,),
        in_specs=[
            pl.BlockSpec((_RB, D), lambda i: (i, 0)),
            pl.BlockSpec((NW, 2, _RB), lambda i: (0, 0, i)),
        ],
        out_specs=[
            pl.BlockSpec((_RB, D), lambda i: (i, 0)),
            pl.BlockSpec((_RB,), lambda i: (i,)),
        ],
        out_shape=[
            jax.ShapeDtypeStruct((N, D), jnp.float32),
            jax.ShapeDtypeStruct((N,), jnp.float32),
        ],
    )(emb, degs)


@functools.partial(
    pl.kernel,
    out_type=jax.ShapeDtypeStruct((NC, N, D), jnp.float32),
    mesh=_mesh,
    scratch_types=[
        pltpu.VMEM_SHARED((N, D), jnp.float32),
        pltpu.VMEM((C,), jnp.int32),
        pltpu.VMEM((C,), jnp.int32),
        pltpu.VMEM((C, D), jnp.float32),
        pltpu.VMEM((ZR, D), jnp.float32),
        pltpu.SemaphoreType.DMA,
    ],
)
def _scatter_kernel(m_hbm, edge_hbm, part_hbm, acc, src_idx, dst_idx, rows, zbuf, sem):
    c = lax.axis_index("c")
    s = lax.axis_index("s")
    wid = c * NS + s
    base = wid * EPW
    zeros = jnp.zeros((L,), jnp.float32)

    def zb_body(i, _):
        def zb_inner(j, _):
            zbuf[i, pl.ds(j * L, L)] = zeros
            return 0

        lax.fori_loop(0, D // L, zb_inner, 0)
        return 0

    lax.fori_loop(0, ZR, zb_body, 0)

    for z in range(NZ):
        pltpu.sync_copy(zbuf, acc.at[pl.ds(s * RPT + z * ZR, ZR)])
    plsc.subcore_barrier()

    def edge_body(k, _):
        pltpu.sync_copy(edge_hbm.at[0, pl.ds(base + k * C, C)], src_idx)
        pltpu.sync_copy(edge_hbm.at[1, pl.ds(base + k * C, C)], dst_idx)
        pltpu.async_copy(m_hbm.at[src_idx], rows, sem).wait()
        pltpu.sync_copy(rows, acc.at[dst_idx], add=True)
        return 0

    lax.fori_loop(0, NCHUNK, edge_body, 0)
    plsc.subcore_barrier()

    for z in range(NZ):
        r0 = s * RPT + z * ZR
        pltpu.sync_copy(acc.at[pl.ds(r0, ZR)], zbuf)
        pltpu.sync_copy(zbuf, part_hbm.at[c, pl.ds(r0, ZR)])


def _final_body(part_ref, innorm_ref, out_ref):
    agg = part_ref[0] + part_ref[1]
    out_ref[...] = agg * (innorm_ref[...] * ALPHA)[:, None]


def _final_call(parts, in_norm):
    return pl.pallas_call(
        _final_body,
        grid=(N // _RB,),
        in_specs=[
            pl.BlockSpec((NC, _RB, D), lambda i: (0, i, 0)),
            pl.BlockSpec((_RB,), lambda i: (i,)),
        ],
        out_specs=pl.BlockSpec((_RB, D), lambda i: (i, 0)),
        out_shape=jax.ShapeDtypeStruct((N, D), jnp.float32),
    )(parts, in_norm)


def kernel(emb, edge_index):
    degs = _deg_kernel(edge_index)
    m, in_norm = _norm_call(emb, degs)
    parts = _scatter_kernel(m, edge_index)
    return _final_call(parts, in_norm)


# speedup vs baseline: 6.6409x; 6.6409x over previous
import functools

import jax
import jax.numpy as jnp
from jax import lax
from jax.experimental import pallas as pl
from jax.experimental.pallas import tpu as pltpu
from jax.experimental.pallas import tpu_sc as plsc

N = 10000
E = 320000
D = 128
N_LAYERS = 3
ALPHA = sum(1.0 / (1 + k) for k in range(N_LAYERS))

NC, NS, L = 2, 16, 16
NW = NC * NS
EPW = E // NW
C = 80
NCHUNK = EPW // C
RPT = N // NS
ZR = 125
NZ = RPT // ZR

_mesh = plsc.VectorSubcoreMesh(
    core_axis_name="c", subcore_axis_name="s", num_cores=NC, num_subcores=NS
)
_sc_params = pltpu.CompilerParams(
    needs_layout_passes=False, use_tc_tiling_on_sc=False
)


@functools.partial(
    pl.kernel,
    out_type=jax.ShapeDtypeStruct((NW, 2, N), jnp.float32),
    mesh=_mesh,
    scratch_types=[
        pltpu.VMEM((EPW,), jnp.int32),
        pltpu.VMEM((N,), jnp.float32),
        pltpu.VMEM((N,), jnp.float32),
    ],
    compiler_params=_sc_params,
)
def _deg_kernel(src_hbm, dst_hbm, deg_hbm, ev, cnt_s, cnt_d):
    c = lax.axis_index("c")
    s = lax.axis_index("s")
    wid = c * NS + s
    base = wid * EPW
    zeros = jnp.zeros((L,), jnp.float32)
    ones = jnp.ones((L,), jnp.float32)

    def zero_body(g, _):
        cnt_s[pl.ds(g * L, L)] = zeros
        cnt_d[pl.ds(g * L, L)] = zeros
        return 0

    lax.fori_loop(0, N // L, zero_body, 0)

    for cnt, eh in ((cnt_s, src_hbm), (cnt_d, dst_hbm)):
        pltpu.sync_copy(eh.at[pl.ds(base, EPW)], ev)

        def acc_body(g, _, cnt=cnt):
            idx = ev[pl.ds(g * L, L)]
            plsc.addupdate_scatter(cnt, [idx], ones)
            return 0

        lax.fori_loop(0, EPW // L, acc_body, 0)

    pltpu.sync_copy(cnt_s, deg_hbm.at[wid, 0])
    pltpu.sync_copy(cnt_d, deg_hbm.at[wid, 1])


def _norm_body(emb_ref, deg_ref, m_ref, innorm_ref):
    out_deg = jnp.sum(deg_ref[:, 0, :], axis=0)
    in_deg = jnp.sum(deg_ref[:, 1, :], axis=0)
    out_norm = lax.rsqrt(jnp.maximum(out_deg, 1.0))
    innorm_ref[...] = lax.rsqrt(jnp.maximum(in_deg, 1.0))
    m_ref[...] = emb_ref[...] * out_norm[:, None]


def _norm_call(emb, degs):
    return pl.pallas_call(
        _norm_body,
        out_shape=[
            jax.ShapeDtypeStruct((N, D), jnp.float32),
            jax.ShapeDtypeStruct((N,), jnp.float32),
        ],
    )(emb, degs)


@functools.partial(
    pl.kernel,
    out_type=jax.ShapeDtypeStruct((NC, N, D), jnp.float32),
    mesh=_mesh,
    scratch_types=[
        pltpu.VMEM_SHARED((N, D), jnp.float32),
        pltpu.VMEM((C,), jnp.int32),
        pltpu.VMEM((C,), jnp.int32),
        pltpu.VMEM((C, D), jnp.float32),
        pltpu.VMEM((ZR, D), jnp.float32),
        pltpu.SemaphoreType.DMA,
    ],
    compiler_params=_sc_params,
)
def _scatter_kernel(m_hbm, src_hbm, dst_hbm, part_hbm, acc, src_idx, dst_idx, rows, zbuf, sem):
    c = lax.axis_index("c")
    s = lax.axis_index("s")
    wid = c * NS + s
    base = wid * EPW
    zeros = jnp.zeros((L,), jnp.float32)

    def zb_body(i, _):
        def zb_inner(j, _):
            zbuf[i, pl.ds(j * L, L)] = zeros
            return 0

        lax.fori_loop(0, D // L, zb_inner, 0)
        return 0

    lax.fori_loop(0, ZR, zb_body, 0)

    for z in range(NZ):
        pltpu.sync_copy(zbuf, acc.at[pl.ds(s * RPT + z * ZR, ZR)])
    plsc.subcore_barrier()

    def edge_body(k, _):
        pltpu.sync_copy(src_hbm.at[pl.ds(base + k * C, C)], src_idx)
        pltpu.sync_copy(dst_hbm.at[pl.ds(base + k * C, C)], dst_idx)
        pltpu.async_copy(m_hbm.at[src_idx], rows, sem).wait()
        pltpu.sync_copy(rows, acc.at[dst_idx], add=True)
        return 0

    lax.fori_loop(0, NCHUNK, edge_body, 0)
    plsc.subcore_barrier()

    for z in range(NZ):
        r0 = s * RPT + z * ZR
        pltpu.sync_copy(acc.at[pl.ds(r0, ZR)], zbuf)
        pltpu.sync_copy(zbuf, part_hbm.at[c, pl.ds(r0, ZR)])


def _final_body(part_ref, innorm_ref, out_ref):
    agg = part_ref[0] + part_ref[1]
    out_ref[...] = agg * (innorm_ref[...] * ALPHA)[:, None]


def _final_call(parts, in_norm):
    return pl.pallas_call(
        _final_body,
        out_shape=jax.ShapeDtypeStruct((N, D), jnp.float32),
    )(parts, in_norm)


def kernel(emb, edge_index):
    src = edge_index[0]
    dst = edge_index[1]
    degs = _deg_kernel(src, dst)
    m, in_norm = _norm_call(emb, degs)
    parts = _scatter_kernel(m, src, dst)
    return _final_call(parts, in_norm)


# speedup vs baseline: 13.5299x; 2.0374x over previous
import functools

import jax
import jax.numpy as jnp
from jax import lax
from jax.experimental import pallas as pl
from jax.experimental.pallas import tpu as pltpu
from jax.experimental.pallas import tpu_sc as plsc

N = 10000
E = 320000
D = 128
N_LAYERS = 3
ALPHA = sum(1.0 / (1 + k) for k in range(N_LAYERS))

NC, NS, L = 2, 16, 16
NW = NC * NS
EPW = E // NW
C = 80
NCHUNK = EPW // C
RPT = N // NS
ZR = 125
NZ = RPT // ZR

_mesh = plsc.VectorSubcoreMesh(
    core_axis_name="c", subcore_axis_name="s", num_cores=NC, num_subcores=NS
)
_sc_params = pltpu.CompilerParams(
    needs_layout_passes=False, use_tc_tiling_on_sc=False
)


@functools.partial(
    pl.kernel,
    out_type=jax.ShapeDtypeStruct((NW, 2, N), jnp.float32),
    mesh=_mesh,
    scratch_types=[
        pltpu.VMEM((EPW,), jnp.int32),
        pltpu.VMEM((N,), jnp.float32),
        pltpu.VMEM((N,), jnp.float32),
    ],
    compiler_params=_sc_params,
)
def _deg_kernel(src_hbm, dst_hbm, deg_hbm, ev, cnt_s, cnt_d):
    c = lax.axis_index("c")
    s = lax.axis_index("s")
    wid = c * NS + s
    base = wid * EPW
    zeros = jnp.zeros((L,), jnp.float32)
    ones = jnp.ones((L,), jnp.float32)

    def zero_body(g, _):
        cnt_s[pl.ds(g * L, L)] = zeros
        cnt_d[pl.ds(g * L, L)] = zeros
        return 0

    lax.fori_loop(0, N // L, zero_body, 0)

    for cnt, eh in ((cnt_s, src_hbm), (cnt_d, dst_hbm)):
        pltpu.sync_copy(eh.at[pl.ds(base, EPW)], ev)

        def acc_body(g, _, cnt=cnt):
            idx = ev[pl.ds(g * L, L)]
            plsc.addupdate_scatter(cnt, [idx], ones)
            return 0

        lax.fori_loop(0, EPW // L, acc_body, 0)

    pltpu.sync_copy(cnt_s, deg_hbm.at[wid, 0])
    pltpu.sync_copy(cnt_d, deg_hbm.at[wid, 1])


def _norm_body(emb_ref, deg_ref, m_ref, innorm_ref):
    out_deg = jnp.sum(deg_ref[:, 0, :], axis=0)
    in_deg = jnp.sum(deg_ref[:, 1, :], axis=0)
    out_norm = lax.rsqrt(jnp.maximum(out_deg, 1.0))
    innorm_ref[...] = lax.rsqrt(jnp.maximum(in_deg, 1.0))
    m_ref[...] = emb_ref[...] * out_norm[:, None]


def _norm_call(emb, degs):
    return pl.pallas_call(
        _norm_body,
        out_shape=[
            jax.ShapeDtypeStruct((N, D), jnp.float32),
            jax.ShapeDtypeStruct((N,), jnp.float32),
        ],
    )(emb, degs)


@functools.partial(
    pl.kernel,
    out_type=jax.ShapeDtypeStruct((NC, N, D), jnp.float32),
    mesh=_mesh,
    scratch_types=[
        pltpu.VMEM_SHARED((N, D), jnp.float32),
        pltpu.VMEM((NCHUNK, C), jnp.int32),
        pltpu.VMEM((NCHUNK, C), jnp.int32),
        pltpu.VMEM((C, D), jnp.float32),
        pltpu.VMEM((C, D), jnp.float32),
        pltpu.SemaphoreType.DMA,
        pltpu.SemaphoreType.DMA,
    ],
    compiler_params=_sc_params,
)
def _scatter_kernel(
    m_hbm, src_hbm, dst_hbm, part_hbm,
    acc, src_all, dst_all, rows0, rows1, sem0, sem1,
):
    c = lax.axis_index("c")
    s = lax.axis_index("s")
    wid = c * NS + s
    zeros = jnp.zeros((L,), jnp.float32)

    pltpu.sync_copy(src_hbm.at[pl.ds(wid * NCHUNK, NCHUNK)], src_all)
    pltpu.sync_copy(dst_hbm.at[pl.ds(wid * NCHUNK, NCHUNK)], dst_all)

    def zb_body(i, _):
        def zb_inner(j, _):
            rows0[i, pl.ds(j * L, L)] = zeros
            return 0

        lax.fori_loop(0, D // L, zb_inner, 0)
        return 0

    lax.fori_loop(0, C, zb_body, 0)

    for q in range((NCHUNK + NS - 1) // NS):
        t = s + NS * q

        @pl.when(t < NCHUNK)
        def _():
            pltpu.sync_copy(rows0, acc.at[pl.ds(t * C, C)])

    plsc.subcore_barrier()

    def gstart(k, buf, sem):
        pltpu.async_copy(m_hbm.at[src_all.at[k]], buf, sem)

    def gwait(buf, sem):
        pltpu.make_async_copy(m_hbm.at[src_all.at[0]], buf, sem).wait()

    def scat(k, buf):
        pltpu.sync_copy(buf, acc.at[dst_all.at[k]], add=True)

    gstart(0, rows0, sem0)

    def pair_body(j, _):
        k = 2 * j
        gstart(k + 1, rows1, sem1)
        gwait(rows0, sem0)
        scat(k, rows0)

        @pl.when(k + 2 < NCHUNK)
        def _():
            gstart(k + 2, rows0, sem0)

        gwait(rows1, sem1)
        scat(k + 1, rows1)
        return 0

    lax.fori_loop(0, NCHUNK // 2, pair_body, 0)
    if NCHUNK % 2 == 1:
        gwait(rows0, sem0)
        scat(NCHUNK - 1, rows0)
    plsc.subcore_barrier()

    for q in range((NCHUNK + NS - 1) // NS):
        t = s + NS * q

        @pl.when(t < NCHUNK)
        def _():
            pltpu.sync_copy(acc.at[pl.ds(t * C, C)], rows0)
            pltpu.sync_copy(rows0, part_hbm.at[c, pl.ds(t * C, C)])


def _final_body(part_ref, innorm_ref, out_ref):
    agg = part_ref[0] + part_ref[1]
    out_ref[...] = agg * (innorm_ref[...] * ALPHA)[:, None]


def _final_call(parts, in_norm):
    return pl.pallas_call(
        _final_body,
        out_shape=jax.ShapeDtypeStruct((N, D), jnp.float32),
    )(parts, in_norm)


def kernel(emb, edge_index):
    src = edge_index[0]
    dst = edge_index[1]
    degs = _deg_kernel(src, dst)
    m, in_norm = _norm_call(emb, degs)
    src2d = src.reshape(E // C, C)
    dst2d = dst.reshape(E // C, C)
    parts = _scatter_kernel(m, src2d, dst2d)
    return _final_call(parts, in_norm)


# speedup vs baseline: 15.0969x; 1.1158x over previous
import functools

import jax
import jax.numpy as jnp
from jax import lax
from jax.experimental import pallas as pl
from jax.experimental.pallas import tpu as pltpu
from jax.experimental.pallas import tpu_sc as plsc

N = 10000
E = 320000
D = 128
N_LAYERS = 3
ALPHA = sum(1.0 / (1 + k) for k in range(N_LAYERS))

NC, NS, L = 2, 16, 16
NW = NC * NS
EPW = E // NW
C = 80
NCHUNK = EPW // C
RPT = N // NS
ZR = 125
NZ = RPT // ZR

_mesh = plsc.VectorSubcoreMesh(
    core_axis_name="c", subcore_axis_name="s", num_cores=NC, num_subcores=NS
)
_sc_params = pltpu.CompilerParams(
    needs_layout_passes=False, use_tc_tiling_on_sc=False
)


@functools.partial(
    pl.kernel,
    out_type=jax.ShapeDtypeStruct((NW, 2, N), jnp.float32),
    mesh=_mesh,
    scratch_types=[
        pltpu.VMEM((EPW,), jnp.int32),
        pltpu.VMEM((N,), jnp.float32),
        pltpu.VMEM((N,), jnp.float32),
    ],
    compiler_params=_sc_params,
)
def _deg_kernel(src_hbm, dst_hbm, deg_hbm, ev, cnt_s, cnt_d):
    c = lax.axis_index("c")
    s = lax.axis_index("s")
    wid = c * NS + s
    base = wid * EPW
    zeros = jnp.zeros((L,), jnp.float32)
    ones = jnp.ones((L,), jnp.float32)

    def zero_body(g, _):
        cnt_s[pl.ds(g * L, L)] = zeros
        cnt_d[pl.ds(g * L, L)] = zeros
        return 0

    lax.fori_loop(0, N // L, zero_body, 0)

    for cnt, eh in ((cnt_s, src_hbm), (cnt_d, dst_hbm)):
        pltpu.sync_copy(eh.at[pl.ds(base, EPW)], ev)

        def acc_body(g, _, cnt=cnt):
            idx = ev[pl.ds(g * L, L)]
            plsc.addupdate_scatter(cnt, [idx], ones)
            return 0

        lax.fori_loop(0, EPW // L, acc_body, 0)

    pltpu.sync_copy(cnt_s, deg_hbm.at[wid, 0])
    pltpu.sync_copy(cnt_d, deg_hbm.at[wid, 1])


def _norm_body(emb_ref, deg_ref, m_ref, innorm_ref):
    out_deg = jnp.sum(deg_ref[:, 0, :], axis=0)
    in_deg = jnp.sum(deg_ref[:, 1, :], axis=0)
    out_norm = lax.rsqrt(jnp.maximum(out_deg, 1.0))
    innorm_ref[...] = lax.rsqrt(jnp.maximum(in_deg, 1.0))
    m_ref[...] = emb_ref[...] * out_norm[:, None]


def _norm_call(emb, degs):
    return pl.pallas_call(
        _norm_body,
        out_shape=[
            jax.ShapeDtypeStruct((N, D), jnp.float32),
            jax.ShapeDtypeStruct((N,), jnp.float32),
        ],
    )(emb, degs)


@functools.partial(
    pl.kernel,
    out_type=jax.ShapeDtypeStruct((NC, N, D), jnp.float32),
    mesh=_mesh,
    scratch_types=[
        pltpu.VMEM_SHARED((N, D), jnp.float32),
        pltpu.VMEM((NCHUNK, C), jnp.int32),
        pltpu.VMEM((NCHUNK, C), jnp.int32),
        pltpu.VMEM((C, D), jnp.float32),
        pltpu.VMEM((C, D), jnp.float32),
        pltpu.VMEM((C, D), jnp.float32),
        pltpu.SemaphoreType.DMA,
        pltpu.SemaphoreType.DMA,
        pltpu.SemaphoreType.DMA,
    ],
    compiler_params=_sc_params,
)
def _scatter_kernel(
    m_hbm, src_hbm, dst_hbm, part_hbm,
    acc, src_all, dst_all, rows0, rows1, rows2, sem0, sem1, sem2,
):
    c = lax.axis_index("c")
    s = lax.axis_index("s")
    wid = c * NS + s
    zeros = jnp.zeros((L,), jnp.float32)

    pltpu.sync_copy(src_hbm.at[pl.ds(wid * NCHUNK, NCHUNK)], src_all)
    pltpu.sync_copy(dst_hbm.at[pl.ds(wid * NCHUNK, NCHUNK)], dst_all)

    def zb_body(i, _):
        def zb_inner(j, _):
            rows0[i, pl.ds(j * L, L)] = zeros
            return 0

        lax.fori_loop(0, D // L, zb_inner, 0)
        return 0

    lax.fori_loop(0, C, zb_body, 0)

    for q in range((NCHUNK + NS - 1) // NS):
        t = s + NS * q

        @pl.when(t < NCHUNK)
        def _():
            pltpu.sync_copy(rows0, acc.at[pl.ds(t * C, C)])

    plsc.subcore_barrier()

    bufs = (rows0, rows1, rows2)
    sems = (sem0, sem1, sem2)
    NB = 3

    def gstart(k, b):
        pltpu.async_copy(m_hbm.at[src_all.at[k]], bufs[b], sems[b])

    def gwait(b):
        pltpu.make_async_copy(m_hbm.at[src_all.at[0]], bufs[b], sems[b]).wait()

    def scat(k, b):
        pltpu.sync_copy(bufs[b], acc.at[dst_all.at[k]], add=True)

    gstart(0, 0)
    gstart(1, 1)

    def ring_body(q, _):
        for r in range(NB):
            k = NB * q + r
            gwait(r)

            @pl.when(k + NB - 1 < NCHUNK)
            def _():
                gstart(k + NB - 1, (r + NB - 1) % NB)

            scat(k, r)
        return 0

    lax.fori_loop(0, NCHUNK // NB, ring_body, 0)
    for k in range((NCHUNK // NB) * NB, NCHUNK):
        gwait(k % NB)
        scat(k, k % NB)
    plsc.subcore_barrier()

    for q in range((NCHUNK + NS - 1) // NS):
        t = s + NS * q

        @pl.when(t < NCHUNK)
        def _():
            pltpu.sync_copy(acc.at[pl.ds(t * C, C)], rows0)
            pltpu.sync_copy(rows0, part_hbm.at[c, pl.ds(t * C, C)])


def _final_body(part_ref, innorm_ref, out_ref):
    agg = part_ref[0] + part_ref[1]
    out_ref[...] = agg * (innorm_ref[...] * ALPHA)[:, None]


def _final_call(parts, in_norm):
    return pl.pallas_call(
        _final_body,
        out_shape=jax.ShapeDtypeStruct((N, D), jnp.float32),
    )(parts, in_norm)


def kernel(emb, edge_index):
    src = edge_index[0]
    dst = edge_index[1]
    degs = _deg_kernel(src, dst)
    m, in_norm = _norm_call(emb, degs)
    src2d = src.reshape(E // C, C)
    dst2d = dst.reshape(E // C, C)
    parts = _scatter_kernel(m, src2d, dst2d)
    return _final_call(parts, in_norm)
